# baseline (speedup 1.0000x reference)
import dataclasses
import functools

import jax
import jax.numpy as jnp
from jax.experimental import pallas as pl
from jax.experimental.pallas import tpu as pltpu
from jax.experimental.pallas import tpu_sc as plsc

_WIN = 128


def _sc_gathers(time_tb, loc_tb, act_tb, ctx_tb, t_idx, l_idx, a_idx, c_idx):
    n = t_idx.shape[0] * t_idx.shape[1]
    nc = c_idx.shape[0] * c_idx.shape[1]
    d = time_tb.shape[1]
    mesh = plsc.VectorSubcoreMesh(core_axis_name="c", subcore_axis_name="s")
    out_type = (
        jax.ShapeDtypeStruct((n, d), jnp.float32),
        jax.ShapeDtypeStruct((n, d), jnp.float32),
        jax.ShapeDtypeStruct((n, d), jnp.float32),
        jax.ShapeDtypeStruct((nc, d), jnp.float32),
    )
    cp = pltpu.CompilerParams()
    if "use_tc_tiling_on_sc" in pltpu.CompilerParams.__dataclass_fields__:
        cp = dataclasses.replace(cp, use_tc_tiling_on_sc=False)

    @functools.partial(pl.kernel, out_type=out_type, mesh=mesh,
                       compiler_params=cp)
    def k(tt_h, lt_h, at_h, ct_h, ti_h, li_h, ai_h, ci_h, to_h, lo_h, ao_h, co_h):
        def body3(ti_v, li_v, ai_v, to_v, lo_v, ao_v):
            pltpu.sync_copy(tt_h.at[ti_v.at[0]], to_v)
            pltpu.sync_copy(lt_h.at[li_v.at[0]], lo_v)
            pltpu.sync_copy(at_h.at[ai_v.at[0]], ao_v)

        pltpu.emit_pipeline(
            body3,
            grid=(n // _WIN,),
            in_specs=[pl.BlockSpec((1, _WIN), lambda i: (i, 0))] * 3,
            out_specs=[pl.BlockSpec((_WIN, d), lambda i: (i, 0))] * 3,
            core_axis_name=("c", "s"),
            dimension_semantics=(pltpu.PARALLEL,),
        )(ti_h, li_h, ai_h, to_h, lo_h, ao_h)

        def bodyc(ci_v, co_v):
            pltpu.sync_copy(ct_h.at[ci_v.at[0]], co_v)

        pltpu.emit_pipeline(
            bodyc,
            grid=(nc // _WIN,),
            in_specs=[pl.BlockSpec((1, _WIN), lambda i: (i, 0))],
            out_specs=[pl.BlockSpec((_WIN, d), lambda i: (i, 0))],
            core_axis_name=("c", "s"),
            dimension_semantics=(pltpu.PARALLEL,),
        )(ci_h, co_h)

    return k(time_tb, loc_tb, act_tb, ctx_tb, t_idx, l_idx, a_idx, c_idx)


def _finish_proj(packed, W, b, rows_per_block):
    half, d2 = packed.shape
    d = d2 // 2
    hb = rows_per_block // 2

    def body(i_ref, w_ref, b_ref, o_ref):
        x = i_ref[...]
        o_ref[0] = jax.lax.dot_general(
            x[:, :d], w_ref[...],
            dimension_numbers=(((1,), (1,)), ((), ())),
            preferred_element_type=jnp.float32,
            precision=jax.lax.Precision.HIGHEST,
        ) + b_ref[...]
        o_ref[1] = jax.lax.dot_general(
            x[:, d:], w_ref[...],
            dimension_numbers=(((1,), (1,)), ((), ())),
            preferred_element_type=jnp.float32,
            precision=jax.lax.Precision.HIGHEST,
        ) + b_ref[...]

    return pl.pallas_call(
        body,
        grid=(half // hb,),
        in_specs=[
            pl.BlockSpec((hb, d2), lambda i: (i, 0)),
            pl.BlockSpec((d, d), lambda i: (0, 0)),
            pl.BlockSpec((1, d), lambda i: (0, 0)),
        ],
        out_specs=pl.BlockSpec((2, hb, d), lambda i: (0, i, 0)),
        out_shape=jax.ShapeDtypeStruct((2, half, d), jnp.float32),
    )(packed, W, b.reshape(1, d))


def _interleave_halves(flat):
    half = flat.shape[0] // 2
    return jnp.stack([flat[:half], flat[half:]], axis=1).reshape(-1)


def kernel(context_tokens, time_tokens, loc_tokens, act_tokens, time_table,
           loc_table, act_table, ctx_tables, W_time, b_time, W_loc, b_loc,
           W_act, b_act):
    B, L = time_tokens.shape
    NF = context_tokens.shape[1]
    ctx_vocab = ctx_tables.shape[1]
    D = time_table.shape[1]
    n = B * L

    ctx_flat = ctx_tables.reshape(NF * ctx_vocab, D)
    c_idx = (context_tokens.astype(jnp.int32)
             + jnp.arange(NF, dtype=jnp.int32)[None, :] * ctx_vocab
             ).reshape(B * NF // _WIN, _WIN)
    t_idx = _interleave_halves(
        time_tokens.astype(jnp.int32).reshape(n)).reshape(n // _WIN, _WIN)
    l_idx = _interleave_halves(
        loc_tokens.astype(jnp.int32).reshape(n)).reshape(n // _WIN, _WIN)
    a_idx = _interleave_halves(
        act_tokens.astype(jnp.int32).reshape(n)).reshape(n // _WIN, _WIN)

    t_raw, l_raw, a_raw, c_raw = _sc_gathers(
        time_table, loc_table, act_table, ctx_flat, t_idx, l_idx, a_idx, c_idx)

    t_out = _finish_proj(t_raw.reshape(n // 2, 2 * D), W_time, b_time,
                         8192).reshape(B, L, D)
    l_out = _finish_proj(l_raw.reshape(n // 2, 2 * D), W_loc, b_loc,
                         8192).reshape(B, L, D)
    a_out = _finish_proj(a_raw.reshape(n // 2, 2 * D), W_act, b_act,
                         8192).reshape(B, L, D)
    c_out = c_raw.reshape(B, NF, D)

    return (c_out, t_out, l_out, a_out)


# speedup vs baseline: 1.3951x; 1.3951x over previous
import dataclasses
import functools

import jax
import jax.numpy as jnp
from jax.experimental import pallas as pl
from jax.experimental.pallas import tpu as pltpu
from jax.experimental.pallas import tpu_sc as plsc

_WIN = 128


def _sc_gathers(time_tb, loc_tb, act_tb, ctx_tb, t_idx, l_idx, a_idx, c_idx):
    n = t_idx.shape[0] * t_idx.shape[1]
    nc = c_idx.shape[0] * c_idx.shape[1]
    d = time_tb.shape[1]
    mesh = plsc.VectorSubcoreMesh(core_axis_name="c", subcore_axis_name="s")
    out_type = (
        jax.ShapeDtypeStruct((n, d), jnp.float32),
        jax.ShapeDtypeStruct((n, d), jnp.float32),
        jax.ShapeDtypeStruct((n, d), jnp.float32),
        jax.ShapeDtypeStruct((nc, d), jnp.float32),
    )
    cp = pltpu.CompilerParams()
    if "use_tc_tiling_on_sc" in pltpu.CompilerParams.__dataclass_fields__:
        cp = dataclasses.replace(cp, use_tc_tiling_on_sc=False)

    @functools.partial(pl.kernel, out_type=out_type, mesh=mesh,
                       compiler_params=cp)
    def k(tt_h, lt_h, at_h, ct_h, ti_h, li_h, ai_h, ci_h, to_h, lo_h, ao_h, co_h):
        def body3(ti_v, li_v, ai_v, to_v, lo_v, ao_v):
            pltpu.sync_copy(tt_h.at[ti_v.at[0]], to_v)
            pltpu.sync_copy(lt_h.at[li_v.at[0]], lo_v)
            pltpu.sync_copy(at_h.at[ai_v.at[0]], ao_v)

        pltpu.emit_pipeline(
            body3,
            grid=(n // _WIN,),
            in_specs=[pl.BlockSpec((1, _WIN), lambda i: (i, 0))] * 3,
            out_specs=[pl.BlockSpec((_WIN, d), lambda i: (i, 0))] * 3,
            core_axis_name=("c", "s"),
            dimension_semantics=(pltpu.PARALLEL,),
        )(ti_h, li_h, ai_h, to_h, lo_h, ao_h)

        def bodyc(ci_v, co_v):
            pltpu.sync_copy(ct_h.at[ci_v.at[0]], co_v)

        pltpu.emit_pipeline(
            bodyc,
            grid=(nc // _WIN,),
            in_specs=[pl.BlockSpec((1, _WIN), lambda i: (i, 0))],
            out_specs=[pl.BlockSpec((_WIN, d), lambda i: (i, 0))],
            core_axis_name=("c", "s"),
            dimension_semantics=(pltpu.PARALLEL,),
        )(ci_h, co_h)

    return k(time_tb, loc_tb, act_tb, ctx_tb, t_idx, l_idx, a_idx, c_idx)


def _finish_proj(packed, W, b, rows_per_block):
    half, d2 = packed.shape
    d = d2 // 2
    hb = rows_per_block // 2
    W2 = jnp.zeros((d2, d2), dtype=W.dtype)
    W2 = W2.at[:d, :d].set(W.T).at[d:, d:].set(W.T)
    b2 = jnp.concatenate([b, b]).reshape(1, d2)

    def body(i_ref, w_ref, b_ref, o_ref):
        y = jax.lax.dot_general(
            i_ref[...], w_ref[...],
            dimension_numbers=(((1,), (0,)), ((), ())),
            preferred_element_type=jnp.float32,
        ) + b_ref[...]
        o_ref[0::2, :] = y[:, :d]
        o_ref[1::2, :] = y[:, d:]

    return pl.pallas_call(
        body,
        grid=(half // hb,),
        in_specs=[
            pl.BlockSpec((hb, d2), lambda i: (i, 0)),
            pl.BlockSpec((d2, d2), lambda i: (0, 0)),
            pl.BlockSpec((1, d2), lambda i: (0, 0)),
        ],
        out_specs=pl.BlockSpec((rows_per_block, d), lambda i: (i, 0)),
        out_shape=jax.ShapeDtypeStruct((half * 2, d), jnp.float32),
    )(packed, W2, b2)


def kernel(context_tokens, time_tokens, loc_tokens, act_tokens, time_table,
           loc_table, act_table, ctx_tables, W_time, b_time, W_loc, b_loc,
           W_act, b_act):
    B, L = time_tokens.shape
    NF = context_tokens.shape[1]
    ctx_vocab = ctx_tables.shape[1]
    D = time_table.shape[1]
    n = B * L

    ctx_flat = ctx_tables.reshape(NF * ctx_vocab, D)
    c_idx = (context_tokens.astype(jnp.int32)
             + jnp.arange(NF, dtype=jnp.int32)[None, :] * ctx_vocab
             ).reshape(B * NF // _WIN, _WIN)
    t_idx = time_tokens.astype(jnp.int32).reshape(n // _WIN, _WIN)
    l_idx = loc_tokens.astype(jnp.int32).reshape(n // _WIN, _WIN)
    a_idx = act_tokens.astype(jnp.int32).reshape(n // _WIN, _WIN)

    t_raw, l_raw, a_raw, c_raw = _sc_gathers(
        time_table, loc_table, act_table, ctx_flat, t_idx, l_idx, a_idx, c_idx)

    t_out = _finish_proj(t_raw.reshape(n // 2, 2 * D), W_time, b_time,
                         8192).reshape(B, L, D)
    l_out = _finish_proj(l_raw.reshape(n // 2, 2 * D), W_loc, b_loc,
                         8192).reshape(B, L, D)
    a_out = _finish_proj(a_raw.reshape(n // 2, 2 * D), W_act, b_act,
                         8192).reshape(B, L, D)
    c_out = c_raw.reshape(B, NF, D)

    return (c_out, t_out, l_out, a_out)


# speedup vs baseline: 1.5465x; 1.1085x over previous
import dataclasses
import functools

import jax
import jax.numpy as jnp
from jax.experimental import pallas as pl
from jax.experimental.pallas import tpu as pltpu
from jax.experimental.pallas import tpu_sc as plsc

_WIN = 128


def _sc_gather(table, idx):
    n = idx.shape[0] * idx.shape[1]
    d = table.shape[1]
    mesh = plsc.VectorSubcoreMesh(core_axis_name="c", subcore_axis_name="s")
    cp = pltpu.CompilerParams()
    if "use_tc_tiling_on_sc" in pltpu.CompilerParams.__dataclass_fields__:
        cp = dataclasses.replace(cp, use_tc_tiling_on_sc=False)

    @functools.partial(
        pl.kernel, out_type=jax.ShapeDtypeStruct((n, d), jnp.float32),
        mesh=mesh, compiler_params=cp)
    def k(tb_h, i_h, o_h):
        def body(i_v, o_v):
            pltpu.sync_copy(tb_h.at[i_v.at[0]], o_v)

        pltpu.emit_pipeline(
            body,
            grid=(n // _WIN,),
            in_specs=[pl.BlockSpec((1, _WIN), lambda i: (i, 0))],
            out_specs=[pl.BlockSpec((_WIN, d), lambda i: (i, 0))],
            core_axis_name=("c", "s"),
            dimension_semantics=(pltpu.PARALLEL,),
        )(i_h, o_h)

    return k(table, idx)


def _finish_proj(packed, W, b, rows_per_block):
    half, d2 = packed.shape
    d = d2 // 2
    hb = rows_per_block // 2
    W2 = jnp.zeros((d2, d2), dtype=W.dtype)
    W2 = W2.at[:d, :d].set(W.T).at[d:, d:].set(W.T)
    b2 = jnp.concatenate([b, b]).reshape(1, d2)

    def body(i_ref, w_ref, b_ref, o_ref):
        y = jax.lax.dot_general(
            i_ref[...], w_ref[...],
            dimension_numbers=(((1,), (0,)), ((), ())),
            preferred_element_type=jnp.float32,
        ) + b_ref[...]
        o_ref[0::2, :] = y[:, :d]
        o_ref[1::2, :] = y[:, d:]

    return pl.pallas_call(
        body,
        grid=(half // hb,),
        in_specs=[
            pl.BlockSpec((hb, d2), lambda i: (i, 0)),
            pl.BlockSpec((d2, d2), lambda i: (0, 0)),
            pl.BlockSpec((1, d2), lambda i: (0, 0)),
        ],
        out_specs=pl.BlockSpec((rows_per_block, d), lambda i: (i, 0)),
        out_shape=jax.ShapeDtypeStruct((half * 2, d), jnp.float32),
    )(packed, W2, b2)


def kernel(context_tokens, time_tokens, loc_tokens, act_tokens, time_table,
           loc_table, act_table, ctx_tables, W_time, b_time, W_loc, b_loc,
           W_act, b_act):
    B, L = time_tokens.shape
    NF = context_tokens.shape[1]
    ctx_vocab = ctx_tables.shape[1]
    D = time_table.shape[1]
    n = B * L

    ctx_flat = ctx_tables.reshape(NF * ctx_vocab, D)
    c_idx = (context_tokens.astype(jnp.int32)
             + jnp.arange(NF, dtype=jnp.int32)[None, :] * ctx_vocab
             ).reshape(B * NF // _WIN, _WIN)
    t_idx = time_tokens.astype(jnp.int32).reshape(n // _WIN, _WIN)
    l_idx = loc_tokens.astype(jnp.int32).reshape(n // _WIN, _WIN)
    a_idx = act_tokens.astype(jnp.int32).reshape(n // _WIN, _WIN)

    c_raw = _sc_gather(ctx_flat, c_idx)
    t_raw = _sc_gather(time_table, t_idx)
    a_raw = _sc_gather(act_table, a_idx)
    nw2 = n // (2 * _WIN)
    l0_raw = _sc_gather(loc_table, l_idx[:nw2])
    l1_raw = _sc_gather(loc_table, l_idx[nw2:])

    t_out = _finish_proj(t_raw.reshape(n // 2, 2 * D), W_time, b_time,
                         8192).reshape(B, L, D)
    a_out = _finish_proj(a_raw.reshape(n // 2, 2 * D), W_act, b_act,
                         8192).reshape(B, L, D)
    l0 = _finish_proj(l0_raw.reshape(n // 4, 2 * D), W_loc, b_loc, 8192)
    l1 = _finish_proj(l1_raw.reshape(n // 4, 2 * D), W_loc, b_loc, 8192)
    l_out = jnp.concatenate(
        [l0.reshape(B // 2, L, D), l1.reshape(B // 2, L, D)], axis=0)
    c_out = c_raw.reshape(B, NF, D)

    return (c_out, t_out, l_out, a_out)


# speedup vs baseline: 1.6898x; 1.0927x over previous
import dataclasses
import functools

import jax
import jax.numpy as jnp
from jax.experimental import pallas as pl
from jax.experimental.pallas import tpu as pltpu
from jax.experimental.pallas import tpu_sc as plsc

_WIN = 128


def _sc_gather(table, idx):
    n = idx.shape[0] * idx.shape[1]
    d = table.shape[1]
    mesh = plsc.VectorSubcoreMesh(core_axis_name="c", subcore_axis_name="s")
    cp = pltpu.CompilerParams()
    if "use_tc_tiling_on_sc" in pltpu.CompilerParams.__dataclass_fields__:
        cp = dataclasses.replace(cp, use_tc_tiling_on_sc=False)

    @functools.partial(
        pl.kernel, out_type=jax.ShapeDtypeStruct((n, d), jnp.float32),
        mesh=mesh, compiler_params=cp)
    def k(tb_h, i_h, o_h):
        def body(i_v, o_v):
            pltpu.sync_copy(tb_h.at[i_v.at[0]], o_v.at[pl.ds(0, _WIN)])
            pltpu.sync_copy(tb_h.at[i_v.at[1]], o_v.at[pl.ds(_WIN, _WIN)])

        pltpu.emit_pipeline(
            body,
            grid=(n // (2 * _WIN),),
            in_specs=[pl.BlockSpec((2, _WIN), lambda i: (i, 0))],
            out_specs=[pl.BlockSpec((2 * _WIN, d), lambda i: (i, 0))],
            core_axis_name=("c", "s"),
            dimension_semantics=(pltpu.PARALLEL,),
        )(i_h, o_h)

    return k(table, idx)


def _finish_ctxT(packed, NF, B, d):
    hb = B // 2

    def body(i_ref, e_ref, o_ref):
        x = i_ref[...]
        o_ref[0, :, :hb] = jax.lax.dot_general(
            e_ref[...], x[:, :d],
            dimension_numbers=(((1,), (1,)), ((), ())),
            preferred_element_type=jnp.float32)
        o_ref[0, :, hb:] = jax.lax.dot_general(
            e_ref[...], x[:, d:],
            dimension_numbers=(((1,), (1,)), ((), ())),
            preferred_element_type=jnp.float32)

    eye = jnp.eye(d, dtype=jnp.float32)
    return pl.pallas_call(
        body,
        grid=(NF,),
        in_specs=[
            pl.BlockSpec((hb, 2 * d), lambda i: (i, 0)),
            pl.BlockSpec((d, d), lambda i: (0, 0)),
        ],
        out_specs=pl.BlockSpec((1, d, B), lambda i: (i, 0, 0)),
        out_shape=jax.ShapeDtypeStruct((NF, d, B), jnp.float32),
    )(packed, eye)


def _finish_proj(packed, W, b, rows_per_block):
    half, d2 = packed.shape
    d = d2 // 2
    hb = rows_per_block // 2
    W2 = jnp.zeros((d2, d2), dtype=W.dtype)
    W2 = W2.at[:d, :d].set(W.T).at[d:, d:].set(W.T)
    b2 = jnp.concatenate([b, b]).reshape(1, d2)

    def body(i_ref, w_ref, b_ref, o_ref):
        y = jax.lax.dot_general(
            i_ref[...], w_ref[...],
            dimension_numbers=(((1,), (0,)), ((), ())),
            preferred_element_type=jnp.float32,
        ) + b_ref[...]
        o_ref[0::2, :] = y[:, :d]
        o_ref[1::2, :] = y[:, d:]

    return pl.pallas_call(
        body,
        grid=(half // hb,),
        in_specs=[
            pl.BlockSpec((hb, d2), lambda i: (i, 0)),
            pl.BlockSpec((d2, d2), lambda i: (0, 0)),
            pl.BlockSpec((1, d2), lambda i: (0, 0)),
        ],
        out_specs=pl.BlockSpec((rows_per_block, d), lambda i: (i, 0)),
        out_shape=jax.ShapeDtypeStruct((half * 2, d), jnp.float32),
    )(packed, W2, b2)


def kernel(context_tokens, time_tokens, loc_tokens, act_tokens, time_table,
           loc_table, act_table, ctx_tables, W_time, b_time, W_loc, b_loc,
           W_act, b_act):
    B, L = time_tokens.shape
    NF = context_tokens.shape[1]
    ctx_vocab = ctx_tables.shape[1]
    D = time_table.shape[1]
    n = B * L

    ctx_flat = ctx_tables.reshape(NF * ctx_vocab, D)
    ctx_T = (context_tokens.astype(jnp.int32).T
             + jnp.arange(NF, dtype=jnp.int32)[:, None] * ctx_vocab)
    c_idx = (ctx_T.reshape(NF, 2, B // 2).transpose(0, 2, 1)
             .reshape(NF * B // _WIN, _WIN))

    rep = 8
    time_rep = jnp.tile(time_table, (rep, 1))
    t_flat = time_tokens.astype(jnp.int32).reshape(n)
    t_spread = t_flat + (jnp.arange(n, dtype=jnp.int32) % rep) * time_table.shape[0]
    t_idx = t_spread.reshape(n // _WIN, _WIN)
    l_idx = loc_tokens.astype(jnp.int32).reshape(n // _WIN, _WIN)
    a_idx = act_tokens.astype(jnp.int32).reshape(n // _WIN, _WIN)

    c_raw = _sc_gather(ctx_flat, c_idx)
    c_pk = c_raw.reshape(NF * B // 2, 2 * D)
    t_raw = _sc_gather(time_rep, t_idx)
    a_raw = _sc_gather(act_table, a_idx)
    l_raw = _sc_gather(loc_table, l_idx)

    c_out = jnp.transpose(_finish_ctxT(c_pk, NF, B, D), (2, 0, 1))
    t_out = _finish_proj(t_raw.reshape(n // 2, 2 * D), W_time, b_time,
                         8192).reshape(B, L, D)
    a_out = _finish_proj(a_raw.reshape(n // 2, 2 * D), W_act, b_act,
                         8192).reshape(B, L, D)
    l_out = _finish_proj(l_raw.reshape(n // 2, 2 * D), W_loc, b_loc,
                         8192).reshape(B, L, D)

    return (c_out, t_out, l_out, a_out)


# speedup vs baseline: 1.6910x; 1.0007x over previous
import dataclasses
import functools

import jax
import jax.numpy as jnp
from jax.experimental import pallas as pl
from jax.experimental.pallas import tpu as pltpu
from jax.experimental.pallas import tpu_sc as plsc

_WIN = 128


def _sc_gather(table, idx):
    n = idx.shape[0] * idx.shape[1]
    d = table.shape[1]
    mesh = plsc.VectorSubcoreMesh(core_axis_name="c", subcore_axis_name="s")
    cp = pltpu.CompilerParams()
    if "use_tc_tiling_on_sc" in pltpu.CompilerParams.__dataclass_fields__:
        cp = dataclasses.replace(cp, use_tc_tiling_on_sc=False)

    @functools.partial(
        pl.kernel, out_type=jax.ShapeDtypeStruct((n, d), jnp.float32),
        mesh=mesh, compiler_params=cp,
        scratch_types=[pltpu.SemaphoreType.DMA])
    def k(tb_h, i_h, o_h, sem):
        def body(i_v, o_v):
            c1 = pltpu.async_copy(tb_h.at[i_v.at[0]], o_v.at[pl.ds(0, _WIN)],
                                  sem)
            c2 = pltpu.async_copy(tb_h.at[i_v.at[1]],
                                  o_v.at[pl.ds(_WIN, _WIN)], sem)
            c1.wait()
            c2.wait()

        pltpu.emit_pipeline(
            body,
            grid=(n // (2 * _WIN),),
            in_specs=[pl.BlockSpec((2, _WIN), lambda i: (i, 0))],
            out_specs=[pl.BlockSpec((2 * _WIN, d), lambda i: (i, 0))],
            core_axis_name=("c", "s"),
            dimension_semantics=(pltpu.PARALLEL,),
        )(i_h, o_h)

    return k(table, idx)


def _sc_gather2(tb1, i1, tb2, i2):
    n1 = i1.shape[0] * i1.shape[1]
    n2 = i2.shape[0] * i2.shape[1]
    d = tb1.shape[1]
    mesh = plsc.VectorSubcoreMesh(core_axis_name="c", subcore_axis_name="s")
    cp = pltpu.CompilerParams()
    if "use_tc_tiling_on_sc" in pltpu.CompilerParams.__dataclass_fields__:
        cp = dataclasses.replace(cp, use_tc_tiling_on_sc=False)
    out_type = (jax.ShapeDtypeStruct((n1, d), jnp.float32),
                jax.ShapeDtypeStruct((n2, d), jnp.float32))

    @functools.partial(pl.kernel, out_type=out_type, mesh=mesh,
                       compiler_params=cp,
                       scratch_types=[pltpu.SemaphoreType.DMA,
                                      pltpu.SemaphoreType.DMA])
    def k(tb1_h, i1_h, tb2_h, i2_h, o1_h, o2_h, s1, s2):
        def body(i1_v, i2_v, o1_v, o2_v):
            cs = [
                pltpu.async_copy(tb1_h.at[i1_v.at[0]],
                                 o1_v.at[pl.ds(0, _WIN)], s1),
                pltpu.async_copy(tb1_h.at[i1_v.at[1]],
                                 o1_v.at[pl.ds(_WIN, _WIN)], s1),
                pltpu.async_copy(tb2_h.at[i2_v.at[0]],
                                 o2_v.at[pl.ds(0, _WIN)], s2),
                pltpu.async_copy(tb2_h.at[i2_v.at[1]],
                                 o2_v.at[pl.ds(_WIN, _WIN)], s2),
            ]
            for c in cs:
                c.wait()

        pltpu.emit_pipeline(
            body,
            grid=(n1 // (2 * _WIN),),
            in_specs=[pl.BlockSpec((2, _WIN), lambda i: (i, 0))] * 2,
            out_specs=[pl.BlockSpec((2 * _WIN, d), lambda i: (i, 0))] * 2,
            core_axis_name=("c", "s"),
            dimension_semantics=(pltpu.PARALLEL,),
        )(i1_h, i2_h, o1_h, o2_h)

    return k(tb1, i1, tb2, i2)


def _finish_ctxT(packed, NF, B, d):
    hb = B // 2

    def body(i_ref, e_ref, o_ref):
        x = i_ref[...]
        o_ref[0, :, :hb] = jax.lax.dot_general(
            e_ref[...], x[:, :d],
            dimension_numbers=(((1,), (1,)), ((), ())),
            preferred_element_type=jnp.float32)
        o_ref[0, :, hb:] = jax.lax.dot_general(
            e_ref[...], x[:, d:],
            dimension_numbers=(((1,), (1,)), ((), ())),
            preferred_element_type=jnp.float32)

    eye = jnp.eye(d, dtype=jnp.float32)
    return pl.pallas_call(
        body,
        grid=(NF,),
        in_specs=[
            pl.BlockSpec((hb, 2 * d), lambda i: (i, 0)),
            pl.BlockSpec((d, d), lambda i: (0, 0)),
        ],
        out_specs=pl.BlockSpec((1, d, B), lambda i: (i, 0, 0)),
        out_shape=jax.ShapeDtypeStruct((NF, d, B), jnp.float32),
    )(packed, eye)


def _finish_proj(packed, W, b, rows_per_block):
    half, d2 = packed.shape
    d = d2 // 2
    hb = rows_per_block // 2
    W2 = jnp.zeros((d2, d2), dtype=W.dtype)
    W2 = W2.at[:d, :d].set(W.T).at[d:, d:].set(W.T)
    b2 = jnp.concatenate([b, b]).reshape(1, d2)

    def body(i_ref, w_ref, b_ref, o_ref):
        y = jax.lax.dot_general(
            i_ref[...], w_ref[...],
            dimension_numbers=(((1,), (0,)), ((), ())),
            preferred_element_type=jnp.float32,
        ) + b_ref[...]
        o_ref[0::2, :] = y[:, :d]
        o_ref[1::2, :] = y[:, d:]

    return pl.pallas_call(
        body,
        grid=(half // hb,),
        in_specs=[
            pl.BlockSpec((hb, d2), lambda i: (i, 0)),
            pl.BlockSpec((d2, d2), lambda i: (0, 0)),
            pl.BlockSpec((1, d2), lambda i: (0, 0)),
        ],
        out_specs=pl.BlockSpec((rows_per_block, d), lambda i: (i, 0)),
        out_shape=jax.ShapeDtypeStruct((half * 2, d), jnp.float32),
    )(packed, W2, b2)


def kernel(context_tokens, time_tokens, loc_tokens, act_tokens, time_table,
           loc_table, act_table, ctx_tables, W_time, b_time, W_loc, b_loc,
           W_act, b_act):
    B, L = time_tokens.shape
    NF = context_tokens.shape[1]
    ctx_vocab = ctx_tables.shape[1]
    D = time_table.shape[1]
    n = B * L

    ctx_flat = ctx_tables.reshape(NF * ctx_vocab, D)
    ctx_T = (context_tokens.astype(jnp.int32).T
             + jnp.arange(NF, dtype=jnp.int32)[:, None] * ctx_vocab)
    c_idx = (ctx_T.reshape(NF, 2, B // 2).transpose(0, 2, 1)
             .reshape(NF * B // _WIN, _WIN))

    rep = 8
    time_rep = jnp.tile(time_table, (rep, 1))
    t_flat = time_tokens.astype(jnp.int32).reshape(n)
    t_spread = t_flat + (jnp.arange(n, dtype=jnp.int32) % rep) * time_table.shape[0]
    t_idx = t_spread.reshape(n // _WIN, _WIN)
    l_idx = loc_tokens.astype(jnp.int32).reshape(n // _WIN, _WIN)
    a_idx = act_tokens.astype(jnp.int32).reshape(n // _WIN, _WIN)

    c_raw = _sc_gather(ctx_flat, c_idx)
    c_pk = c_raw.reshape(NF * B // 2, 2 * D)
    t_raw, a_raw = _sc_gather2(time_rep, t_idx, act_table, a_idx)
    l_raw = _sc_gather(loc_table, l_idx)

    c_out = jnp.transpose(_finish_ctxT(c_pk, NF, B, D), (2, 0, 1))
    t_out = _finish_proj(t_raw.reshape(n // 2, 2 * D), W_time, b_time,
                         8192).reshape(B, L, D)
    a_out = _finish_proj(a_raw.reshape(n // 2, 2 * D), W_act, b_act,
                         8192).reshape(B, L, D)
    l_out = _finish_proj(l_raw.reshape(n // 2, 2 * D), W_loc, b_loc,
                         8192).reshape(B, L, D)

    return (c_out, t_out, l_out, a_out)


# speedup vs baseline: 1.7066x; 1.0092x over previous
import dataclasses
import functools

import jax
import jax.numpy as jnp
from jax.experimental import pallas as pl
from jax.experimental.pallas import tpu as pltpu
from jax.experimental.pallas import tpu_sc as plsc

_WIN = 128


def _sc_gather(table, idx):
    n = idx.shape[0] * idx.shape[1]
    d = table.shape[1]
    mesh = plsc.VectorSubcoreMesh(core_axis_name="c", subcore_axis_name="s")
    cp = pltpu.CompilerParams()
    if "use_tc_tiling_on_sc" in pltpu.CompilerParams.__dataclass_fields__:
        cp = dataclasses.replace(cp, use_tc_tiling_on_sc=False)

    @functools.partial(
        pl.kernel, out_type=jax.ShapeDtypeStruct((n, d), jnp.float32),
        mesh=mesh, compiler_params=cp,
        scratch_types=[pltpu.SemaphoreType.DMA])
    def k(tb_h, i_h, o_h, sem):
        def body(i_v, o_v):
            c1 = pltpu.async_copy(tb_h.at[i_v.at[0]], o_v.at[pl.ds(0, _WIN)],
                                  sem)
            c2 = pltpu.async_copy(tb_h.at[i_v.at[1]],
                                  o_v.at[pl.ds(_WIN, _WIN)], sem)
            c1.wait()
            c2.wait()

        pltpu.emit_pipeline(
            body,
            grid=(n // (2 * _WIN),),
            in_specs=[pl.BlockSpec((2, _WIN), lambda i: (i, 0))],
            out_specs=[pl.BlockSpec((2 * _WIN, d), lambda i: (i, 0))],
            core_axis_name=("c", "s"),
            dimension_semantics=(pltpu.PARALLEL,),
        )(i_h, o_h)

    return k(table, idx)


def _sc_gather2(tb1, i1, tb2, i2):
    n1 = i1.shape[0] * i1.shape[1]
    n2 = i2.shape[0] * i2.shape[1]
    d = tb1.shape[1]
    mesh = plsc.VectorSubcoreMesh(core_axis_name="c", subcore_axis_name="s")
    cp = pltpu.CompilerParams()
    if "use_tc_tiling_on_sc" in pltpu.CompilerParams.__dataclass_fields__:
        cp = dataclasses.replace(cp, use_tc_tiling_on_sc=False)
    out_type = (jax.ShapeDtypeStruct((n1, d), jnp.float32),
                jax.ShapeDtypeStruct((n2, d), jnp.float32))

    @functools.partial(pl.kernel, out_type=out_type, mesh=mesh,
                       compiler_params=cp,
                       scratch_types=[pltpu.SemaphoreType.DMA,
                                      pltpu.SemaphoreType.DMA])
    def k(tb1_h, i1_h, tb2_h, i2_h, o1_h, o2_h, s1, s2):
        def body(i1_v, i2_v, o1_v, o2_v):
            cs = [
                pltpu.async_copy(tb1_h.at[i1_v.at[0]],
                                 o1_v.at[pl.ds(0, _WIN)], s1),
                pltpu.async_copy(tb1_h.at[i1_v.at[1]],
                                 o1_v.at[pl.ds(_WIN, _WIN)], s1),
                pltpu.async_copy(tb2_h.at[i2_v.at[0]],
                                 o2_v.at[pl.ds(0, _WIN)], s2),
                pltpu.async_copy(tb2_h.at[i2_v.at[1]],
                                 o2_v.at[pl.ds(_WIN, _WIN)], s2),
            ]
            for c in cs:
                c.wait()

        pltpu.emit_pipeline(
            body,
            grid=(n1 // (2 * _WIN),),
            in_specs=[pl.BlockSpec((2, _WIN), lambda i: (i, 0))] * 2,
            out_specs=[pl.BlockSpec((2 * _WIN, d), lambda i: (i, 0))] * 2,
            core_axis_name=("c", "s"),
            dimension_semantics=(pltpu.PARALLEL,),
        )(i1_h, i2_h, o1_h, o2_h)

    return k(tb1, i1, tb2, i2)


def _finish_ctxT(packed, NF, B, d):
    hb = B // 2

    def body(i_ref, e_ref, o_ref):
        x = i_ref[...]
        o_ref[0, :, :hb] = jax.lax.dot_general(
            e_ref[...], x[:, :d],
            dimension_numbers=(((1,), (1,)), ((), ())),
            preferred_element_type=jnp.float32)
        o_ref[0, :, hb:] = jax.lax.dot_general(
            e_ref[...], x[:, d:],
            dimension_numbers=(((1,), (1,)), ((), ())),
            preferred_element_type=jnp.float32)

    eye = jnp.eye(d, dtype=jnp.float32)
    return pl.pallas_call(
        body,
        grid=(NF,),
        in_specs=[
            pl.BlockSpec((hb, 2 * d), lambda i: (i, 0)),
            pl.BlockSpec((d, d), lambda i: (0, 0)),
        ],
        out_specs=pl.BlockSpec((1, d, B), lambda i: (i, 0, 0)),
        out_shape=jax.ShapeDtypeStruct((NF, d, B), jnp.float32),
    )(packed, eye)


def _finish_proj(packed, W, b, rows_per_block):
    half, d2 = packed.shape
    d = d2 // 2
    hb = rows_per_block // 2
    W2 = jnp.zeros((d2, d2), dtype=W.dtype)
    W2 = W2.at[:d, :d].set(W.T).at[d:, d:].set(W.T)
    b2 = jnp.concatenate([b, b]).reshape(1, d2)

    def body(i_ref, w_ref, b_ref, o_ref):
        y = jax.lax.dot_general(
            i_ref[...], w_ref[...],
            dimension_numbers=(((1,), (0,)), ((), ())),
            preferred_element_type=jnp.float32,
        ) + b_ref[...]
        o_ref[0::2, :] = y[:, :d]
        o_ref[1::2, :] = y[:, d:]

    return pl.pallas_call(
        body,
        grid=(half // hb,),
        in_specs=[
            pl.BlockSpec((hb, d2), lambda i: (i, 0)),
            pl.BlockSpec((d2, d2), lambda i: (0, 0)),
            pl.BlockSpec((1, d2), lambda i: (0, 0)),
        ],
        out_specs=pl.BlockSpec((rows_per_block, d), lambda i: (i, 0)),
        out_shape=jax.ShapeDtypeStruct((half * 2, d), jnp.float32),
    )(packed, W2, b2)


def kernel(context_tokens, time_tokens, loc_tokens, act_tokens, time_table,
           loc_table, act_table, ctx_tables, W_time, b_time, W_loc, b_loc,
           W_act, b_act):
    B, L = time_tokens.shape
    NF = context_tokens.shape[1]
    ctx_vocab = ctx_tables.shape[1]
    D = time_table.shape[1]
    n = B * L

    ctx_flat = ctx_tables.reshape(NF * ctx_vocab, D)
    ctx_T = (context_tokens.astype(jnp.int32).T
             + jnp.arange(NF, dtype=jnp.int32)[:, None] * ctx_vocab)
    c_idx = (ctx_T.reshape(NF, 2, B // 2).transpose(0, 2, 1)
             .reshape(NF * B // _WIN, _WIN))

    rep = 8
    time_rep = jnp.tile(time_table, (rep, 1))
    t_flat = time_tokens.astype(jnp.int32).reshape(n)
    t_spread = t_flat + (jnp.arange(n, dtype=jnp.int32) % rep) * time_table.shape[0]
    t_idx = t_spread.reshape(n // _WIN, _WIN)
    l_idx = loc_tokens.astype(jnp.int32).reshape(n // _WIN, _WIN)
    a_idx = act_tokens.astype(jnp.int32).reshape(n // _WIN, _WIN)

    c_raw = _sc_gather(ctx_flat, c_idx)
    c_pk = c_raw.reshape(NF * B // 2, 2 * D)
    t_raw, a_raw = _sc_gather2(time_rep, t_idx, act_table, a_idx)
    l_raw = _sc_gather(loc_table, l_idx)

    c_out = jnp.transpose(_finish_ctxT(c_pk, NF, B, D), (2, 0, 1))
    t_out = _finish_proj(t_raw.reshape(n // 2, 2 * D), W_time, b_time,
                         16384).reshape(B, L, D)
    a_out = _finish_proj(a_raw.reshape(n // 2, 2 * D), W_act, b_act,
                         16384).reshape(B, L, D)
    l_out = _finish_proj(l_raw.reshape(n // 2, 2 * D), W_loc, b_loc,
                         16384).reshape(B, L, D)

    return (c_out, t_out, l_out, a_out)
